# baseline (speedup 1.0000x reference)
import jax
import jax.numpy as jnp
from jax import lax
from jax.experimental import pallas as pl
from jax.experimental.pallas import tpu as pltpu
from jax.experimental.pallas import tpu_sc as plsc

N = 10000
E = 320000
D = 128
NC = 2
NS = 16
LANES = 16
EPT = E // NS
CB = 80
CHUNKS = EPT // CB
NPAD = 10240
ROWS_PT = NPAD // NS


def _mm_body(m_ref, x_ref, w0_ref, w1_ref, t_ref, c_ref):
    x = x_ref[...]
    n0 = jnp.dot(x, w0_ref[...], preferred_element_type=jnp.float32)
    n1 = jnp.dot(x, w1_ref[...], preferred_element_type=jnp.float32)
    third = jnp.float32(1.0 / 3.0)
    t_ref[0] = n0 * (m_ref[0] * third)
    t_ref[1] = n1 * (m_ref[1] * third)
    c_ref[...] = n1 * (m_ref[2] * third)


def _matmuls(x, W0, W1, merger):
    bm = 1000
    return pl.pallas_call(
        _mm_body,
        grid=(N // bm,),
        in_specs=[
            pl.BlockSpec(memory_space=pltpu.SMEM),
            pl.BlockSpec((bm, D), lambda i: (i, 0)),
            pl.BlockSpec((D, D), lambda i: (0, 0)),
            pl.BlockSpec((D, D), lambda i: (0, 0)),
        ],
        out_specs=[
            pl.BlockSpec((2, bm, D), lambda i: (0, i, 0)),
            pl.BlockSpec((bm, D), lambda i: (i, 0)),
        ],
        out_shape=[
            jax.ShapeDtypeStruct((2, N, D), jnp.float32),
            jax.ShapeDtypeStruct((N, D), jnp.float32),
        ],
    )(merger, x, W0, W1)


def _sc_body(t_hbm, sd_hbm, we_hbm, p_hbm,
             sd_v, w_v, rows_v, scl_v, acc,
             sem_g0, sem_g1, sem_s0, sem_s1, sem_i0, sem_i1):
    cc = lax.axis_index("c")
    s = lax.axis_index("s")
    tbase = s * EPT

    zvec = jnp.zeros((LANES,), jnp.float32)

    def _zrow(i, carry):
        for j in range(D // LANES):
            scl_v[0, i, pl.ds(j * LANES, LANES)] = zvec
        return carry

    lax.fori_loop(0, CB, _zrow, 0)
    for k in range(ROWS_PT // CB):
        pltpu.sync_copy(scl_v.at[0], acc.at[pl.ds(s * ROWS_PT + k * CB, CB)])

    plsc.subcore_barrier()

    gsems = (sem_g0, sem_g1)
    ssems = (sem_s0, sem_s1)
    isems = (sem_i0, sem_i1)
    NROW = NC * E // CB

    def _stage_start(g, sdslot, wslot, isem):
        grow = cc * (NROW // NC) + s * CHUNKS + g
        base = cc * E + tbase + g * CB
        pltpu.async_copy(sd_hbm.at[grow], sd_v.at[sdslot], isem)
        pltpu.async_copy(we_hbm.at[pl.ds(base, CB)], w_v.at[wslot], isem)

    def _stage_wait(sdslot, wslot, isem):
        pltpu.make_async_copy(sd_hbm.at[0], sd_v.at[sdslot], isem).wait()
        pltpu.make_async_copy(we_hbm.at[pl.ds(tbase, CB)], w_v.at[wslot],
                              isem).wait()

    def _compute(slot):
        for grp in range(CB // LANES):
            w16 = w_v[slot, pl.ds(grp * LANES, LANES)]
            for e in range(LANES):
                wb = w16[e]
                ee = grp * LANES + e
                for j in range(D // LANES):
                    scl_v[slot, ee, pl.ds(j * LANES, LANES)] = (
                        rows_v[slot, ee, pl.ds(j * LANES, LANES)] * wb)

    tbl = t_hbm.at[cc]

    _stage_start(0, 0, 0, sem_i0)
    _stage_start(1, 1, 1, sem_i1)
    _stage_wait(0, 0, sem_i0)
    pltpu.async_copy(tbl.at[sd_v.at[0, 0]], rows_v.at[0], sem_g0)

    def _half(t, par, slot4):
        g = 2 * t + par
        @pl.when(t > 0)
        def _():
            pltpu.make_async_copy(
                scl_v.at[par], acc.at[sd_v.at[lax.rem(g + 2, 4), 1]],
                ssems[par]).wait()
        nxt = g + 1
        @pl.when(nxt < CHUNKS)
        def _():
            _stage_wait(lax.rem(nxt, 4), 1 - par, isems[1 - par])
            pltpu.async_copy(tbl.at[sd_v.at[lax.rem(nxt, 4), 0]],
                             rows_v.at[1 - par], gsems[1 - par])
        pltpu.make_async_copy(tbl.at[sd_v.at[slot4, 0]], rows_v.at[par],
                              gsems[par]).wait()
        _compute(par)
        nn = g + 2
        @pl.when(nn < CHUNKS)
        def _():
            _stage_start(nn, lax.rem(nn, 4), par, isems[par])
        pltpu.async_copy(scl_v.at[par], acc.at[sd_v.at[slot4, 1]], ssems[par],
                         add=True)

    def _body(t, carry):
        a = 2 * t
        _half(t, 0, lax.rem(a, 4))
        _half(t, 1, lax.rem(a + 1, 4))
        return carry

    lax.fori_loop(0, CHUNKS // 2, _body, 0)

    pltpu.make_async_copy(scl_v.at[0], acc.at[sd_v.at[0, 1]], sem_s0).wait()
    pltpu.make_async_copy(scl_v.at[1], acc.at[sd_v.at[1, 1]], sem_s1).wait()

    plsc.subcore_barrier()

    off = s * ROWS_PT
    pltpu.sync_copy(acc.at[pl.ds(off, ROWS_PT)],
                    p_hbm.at[cc, pl.ds(off, ROWS_PT)])


def _sc_aggregate(T, sd, we):
    mesh = plsc.VectorSubcoreMesh(
        core_axis_name="c", subcore_axis_name="s", num_cores=NC, num_subcores=NS)
    f = pl.kernel(
        _sc_body,
        out_type=jax.ShapeDtypeStruct((2, NPAD, D), jnp.float32),
        mesh=mesh,
        scratch_types=[
            pltpu.VMEM((4, 2, CB), jnp.int32),
            pltpu.VMEM((2, CB), jnp.float32),
            pltpu.VMEM((2, CB, D), jnp.float32),
            pltpu.VMEM((2, CB, D), jnp.float32),
            pltpu.VMEM_SHARED((NPAD, D), jnp.float32),
            pltpu.SemaphoreType.DMA,
            pltpu.SemaphoreType.DMA,
            pltpu.SemaphoreType.DMA,
            pltpu.SemaphoreType.DMA,
            pltpu.SemaphoreType.DMA,
            pltpu.SemaphoreType.DMA,
        ],
    )
    return f(T, sd, we)


def _combine_body(p0_ref, p1_ref, c_ref, o_ref):
    o_ref[...] = p0_ref[0] + p1_ref[0] + c_ref[...]


def _combine(P, C):
    bm = 1000
    return pl.pallas_call(
        _combine_body,
        grid=(N // bm,),
        in_specs=[
            pl.BlockSpec((1, bm, D), lambda i: (0, i, 0)),
            pl.BlockSpec((1, bm, D), lambda i: (1, i, 0)),
            pl.BlockSpec((bm, D), lambda i: (i, 0)),
        ],
        out_specs=pl.BlockSpec((bm, D), lambda i: (i, 0)),
        out_shape=jax.ShapeDtypeStruct((N, D), jnp.float32),
    )(P, P, C)


def kernel(x, g1_src, g1_dst, g1_w, g2_src, g2_dst, g2_w, W0, W1, merger):
    T, C = _matmuls(x, W0, W1, merger)
    rows = E // CB
    sd = jnp.concatenate([
        jnp.stack([g1_src.reshape(rows, CB), g1_dst.reshape(rows, CB)], axis=1),
        jnp.stack([g2_src.reshape(rows, CB), g2_dst.reshape(rows, CB)], axis=1),
    ])
    we = jnp.concatenate([g1_w, g2_w])
    P = _sc_aggregate(T, sd, we)
    return _combine(P, C)


# speedup vs baseline: 1.3517x; 1.3517x over previous
import jax
import jax.numpy as jnp
from jax import lax
from jax.experimental import pallas as pl
from jax.experimental.pallas import tpu as pltpu
from jax.experimental.pallas import tpu_sc as plsc

N = 10000
E = 320000
D = 128
NC = 2
NS = 16
LANES = 16
EPT = E // NS
CB = 80
CHUNKS = EPT // CB
NPAD = 10240
ROWS_PT = NPAD // NS


def _mm_body(m_ref, x_ref, w0_ref, w1_ref, t_ref, c_ref):
    x = x_ref[...]
    n0 = jnp.dot(x, w0_ref[...], preferred_element_type=jnp.float32)
    n1 = jnp.dot(x, w1_ref[...], preferred_element_type=jnp.float32)
    third = jnp.float32(1.0 / 3.0)
    t_ref[0] = n0 * (m_ref[0] * third)
    t_ref[1] = n1 * (m_ref[1] * third)
    c_ref[...] = n1 * (m_ref[2] * third)


def _matmuls(x, W0, W1, merger):
    bm = 1000
    return pl.pallas_call(
        _mm_body,
        grid=(N // bm,),
        in_specs=[
            pl.BlockSpec(memory_space=pltpu.SMEM),
            pl.BlockSpec((bm, D), lambda i: (i, 0)),
            pl.BlockSpec((D, D), lambda i: (0, 0)),
            pl.BlockSpec((D, D), lambda i: (0, 0)),
        ],
        out_specs=[
            pl.BlockSpec((2, bm, D), lambda i: (0, i, 0)),
            pl.BlockSpec((bm, D), lambda i: (i, 0)),
        ],
        out_shape=[
            jax.ShapeDtypeStruct((2, N, D), jnp.float32),
            jax.ShapeDtypeStruct((N, D), jnp.float32),
        ],
    )(merger, x, W0, W1)


def _sc_body(t_hbm, se_hbm, de_hbm, we_hbm, p_hbm,
             src_v, dst_v, w_v, rows_v, acc,
             sem_g0, sem_g1, sem_s0, sem_s1, sem_i0, sem_i1):
    cc = lax.axis_index("c")
    s = lax.axis_index("s")
    tbase = s * EPT

    zvec = jnp.zeros((LANES,), jnp.float32)

    def _zrow(i, carry):
        for j in range(D // LANES):
            rows_v[0, i, pl.ds(j * LANES, LANES)] = zvec
        return carry

    lax.fori_loop(0, CB, _zrow, 0)
    for k in range(ROWS_PT // CB):
        pltpu.sync_copy(rows_v.at[0], acc.at[pl.ds(s * ROWS_PT + k * CB, CB)])

    plsc.subcore_barrier()

    gsems = (sem_g0, sem_g1)
    ssems = (sem_s0, sem_s1)
    isems = (sem_i0, sem_i1)

    def _stage_start(g, isem):
        base = cc * E + tbase + g * CB
        pltpu.async_copy(se_hbm.at[pl.ds(base, CB)], src_v.at[lax.rem(g, 4)],
                         isem)
        pltpu.async_copy(de_hbm.at[pl.ds(base, CB)], dst_v.at[lax.rem(g, 8)],
                         isem)
        pltpu.async_copy(we_hbm.at[pl.ds(base, CB)], w_v.at[lax.rem(g, 4)],
                         isem)

    def _stage_wait(isem):
        pltpu.make_async_copy(se_hbm.at[pl.ds(tbase, CB)], src_v.at[0],
                              isem).wait()
        pltpu.make_async_copy(de_hbm.at[pl.ds(tbase, CB)], dst_v.at[0],
                              isem).wait()
        pltpu.make_async_copy(we_hbm.at[pl.ds(tbase, CB)], w_v.at[0],
                              isem).wait()

    def _compute(r4, w4):
        for grp in range(CB // LANES):
            w16 = w_v[w4, pl.ds(grp * LANES, LANES)]
            for e in range(LANES):
                wb = w16[e]
                ee = grp * LANES + e
                for j in range(D // LANES):
                    rows_v[r4, ee, pl.ds(j * LANES, LANES)] = (
                        rows_v[r4, ee, pl.ds(j * LANES, LANES)] * wb)

    tbl = t_hbm.at[cc]

    for g0 in range(4):
        _stage_start(g0, isems[g0 % 2])
    _stage_wait(sem_i0)
    pltpu.async_copy(tbl.at[src_v.at[0]], rows_v.at[0], sem_g0)
    _stage_wait(sem_i1)
    pltpu.async_copy(tbl.at[src_v.at[1]], rows_v.at[1], sem_g1)

    def _chunk(g, carry):
        par = lax.rem(g, 2)
        r4 = lax.rem(g, 4)
        d8 = lax.rem(g, 8)
        @pl.when(g > 1)
        def _():
            @pl.when(par == 0)
            def _():
                pltpu.make_async_copy(
                    rows_v.at[lax.rem(g + 2, 4)],
                    acc.at[dst_v.at[lax.rem(g + 6, 8)]], sem_s0).wait()
            @pl.when(par == 1)
            def _():
                pltpu.make_async_copy(
                    rows_v.at[lax.rem(g + 2, 4)],
                    acc.at[dst_v.at[lax.rem(g + 6, 8)]], sem_s1).wait()
        @pl.when(par == 0)
        def _():
            pltpu.make_async_copy(tbl.at[src_v.at[r4]], rows_v.at[r4],
                                  sem_g0).wait()
            @pl.when(g + 2 < CHUNKS)
            def _():
                _stage_wait(sem_i0)
                pltpu.async_copy(tbl.at[src_v.at[lax.rem(g + 2, 4)]],
                                 rows_v.at[lax.rem(g + 2, 4)], sem_g0)
        @pl.when(par == 1)
        def _():
            pltpu.make_async_copy(tbl.at[src_v.at[r4]], rows_v.at[r4],
                                  sem_g1).wait()
            @pl.when(g + 2 < CHUNKS)
            def _():
                _stage_wait(sem_i1)
                pltpu.async_copy(tbl.at[src_v.at[lax.rem(g + 2, 4)]],
                                 rows_v.at[lax.rem(g + 2, 4)], sem_g1)
        _compute(r4, r4)
        @pl.when(g + 4 < CHUNKS)
        def _():
            @pl.when(par == 0)
            def _():
                _stage_start(g + 4, sem_i0)
            @pl.when(par == 1)
            def _():
                _stage_start(g + 4, sem_i1)
        @pl.when(par == 0)
        def _():
            pltpu.async_copy(rows_v.at[r4], acc.at[dst_v.at[d8]], sem_s0,
                             add=True)
        @pl.when(par == 1)
        def _():
            pltpu.async_copy(rows_v.at[r4], acc.at[dst_v.at[d8]], sem_s1,
                             add=True)
        return carry

    lax.fori_loop(0, CHUNKS, _chunk, 0)

    pltpu.make_async_copy(rows_v.at[0], acc.at[dst_v.at[0]], sem_s0).wait()
    pltpu.make_async_copy(rows_v.at[1], acc.at[dst_v.at[1]], sem_s1).wait()

    plsc.subcore_barrier()

    off = s * ROWS_PT
    pltpu.sync_copy(acc.at[pl.ds(off, ROWS_PT)],
                    p_hbm.at[cc, pl.ds(off, ROWS_PT)])


def _sc_aggregate(T, se, de, we):
    mesh = plsc.VectorSubcoreMesh(
        core_axis_name="c", subcore_axis_name="s", num_cores=NC, num_subcores=NS)
    f = pl.kernel(
        _sc_body,
        out_type=jax.ShapeDtypeStruct((2, NPAD, D), jnp.float32),
        mesh=mesh,
        scratch_types=[
            pltpu.VMEM((4, CB), jnp.int32),
            pltpu.VMEM((8, CB), jnp.int32),
            pltpu.VMEM((4, CB), jnp.float32),
            pltpu.VMEM((4, CB, D), jnp.float32),
            pltpu.VMEM_SHARED((NPAD, D), jnp.float32),
            pltpu.SemaphoreType.DMA,
            pltpu.SemaphoreType.DMA,
            pltpu.SemaphoreType.DMA,
            pltpu.SemaphoreType.DMA,
            pltpu.SemaphoreType.DMA,
            pltpu.SemaphoreType.DMA,
        ],
    )
    return f(T, se, de, we)


def _combine_body(p0_ref, p1_ref, c_ref, o_ref):
    o_ref[...] = p0_ref[0] + p1_ref[0] + c_ref[...]


def _combine(P, C):
    bm = 1000
    return pl.pallas_call(
        _combine_body,
        grid=(N // bm,),
        in_specs=[
            pl.BlockSpec((1, bm, D), lambda i: (0, i, 0)),
            pl.BlockSpec((1, bm, D), lambda i: (1, i, 0)),
            pl.BlockSpec((bm, D), lambda i: (i, 0)),
        ],
        out_specs=pl.BlockSpec((bm, D), lambda i: (i, 0)),
        out_shape=jax.ShapeDtypeStruct((N, D), jnp.float32),
    )(P, P, C)


def kernel(x, g1_src, g1_dst, g1_w, g2_src, g2_dst, g2_w, W0, W1, merger):
    T, C = _matmuls(x, W0, W1, merger)
    se = jnp.concatenate([g1_src, g2_src])
    de = jnp.concatenate([g1_dst, g2_dst])
    we = jnp.concatenate([g1_w, g2_w])
    P = _sc_aggregate(T, se, de, we)
    return _combine(P, C)


# speedup vs baseline: 1.3580x; 1.0047x over previous
import jax
import jax.numpy as jnp
from jax import lax
from jax.experimental import pallas as pl
from jax.experimental.pallas import tpu as pltpu
from jax.experimental.pallas import tpu_sc as plsc

N = 10000
E = 320000
D = 128
NC = 2
NS = 16
LANES = 16
EPT = E // NS
CB = 80
CHUNKS = EPT // CB
NPAD = 10240
ROWS_PT = NPAD // NS


def _mm_body(m_ref, x_ref, w0_ref, w1_ref, t_ref, c_ref):
    x = x_ref[...]
    n0 = jnp.dot(x, w0_ref[...], preferred_element_type=jnp.float32)
    n1 = jnp.dot(x, w1_ref[...], preferred_element_type=jnp.float32)
    third = jnp.float32(1.0 / 3.0)
    t_ref[0] = n0 * (m_ref[0] * third)
    t_ref[1] = n1 * (m_ref[1] * third)
    c_ref[...] = n1 * (m_ref[2] * third)


def _matmuls(x, W0, W1, merger):
    bm = 1000
    return pl.pallas_call(
        _mm_body,
        grid=(N // bm,),
        in_specs=[
            pl.BlockSpec(memory_space=pltpu.SMEM),
            pl.BlockSpec((bm, D), lambda i: (i, 0)),
            pl.BlockSpec((D, D), lambda i: (0, 0)),
            pl.BlockSpec((D, D), lambda i: (0, 0)),
        ],
        out_specs=[
            pl.BlockSpec((2, bm, D), lambda i: (0, i, 0)),
            pl.BlockSpec((bm, D), lambda i: (i, 0)),
        ],
        out_shape=[
            jax.ShapeDtypeStruct((2, N, D), jnp.float32),
            jax.ShapeDtypeStruct((N, D), jnp.float32),
        ],
    )(merger, x, W0, W1)


def _sc_body(t_hbm, se_hbm, de_hbm, we_hbm, p_hbm,
             src_v, dst_v, w_v, rows_v, acc,
             sem_g0, sem_g1, sem_s0, sem_s1, sem_i0, sem_i1):
    cc = lax.axis_index("c")
    s = lax.axis_index("s")
    tbase = s * EPT

    zvec = jnp.zeros((LANES,), jnp.float32)

    def _zrow(i, carry):
        for j in range(D // LANES):
            rows_v[0, i, pl.ds(j * LANES, LANES)] = zvec
        return carry

    lax.fori_loop(0, CB, _zrow, 0)
    for k in range(ROWS_PT // CB):
        pltpu.sync_copy(rows_v.at[0], acc.at[pl.ds(s * ROWS_PT + k * CB, CB)])

    plsc.subcore_barrier()

    gsems = (sem_g0, sem_g1)
    ssems = (sem_s0, sem_s1)
    isems = (sem_i0, sem_i1)

    def _stage_start(g, isem):
        base = cc * E + tbase + g * CB
        pltpu.async_copy(se_hbm.at[pl.ds(base, CB)], src_v.at[lax.rem(g, 4)],
                         isem)
        pltpu.async_copy(de_hbm.at[pl.ds(base, CB)], dst_v.at[lax.rem(g, 8)],
                         isem)
        pltpu.async_copy(we_hbm.at[pl.ds(base, CB)], w_v.at[lax.rem(g, 4)],
                         isem)

    def _stage_wait(isem):
        pltpu.make_async_copy(se_hbm.at[pl.ds(tbase, CB)], src_v.at[0],
                              isem).wait()
        pltpu.make_async_copy(de_hbm.at[pl.ds(tbase, CB)], dst_v.at[0],
                              isem).wait()
        pltpu.make_async_copy(we_hbm.at[pl.ds(tbase, CB)], w_v.at[0],
                              isem).wait()

    def _compute(r4, w4):
        for grp in range(CB // LANES):
            w16 = w_v[w4, pl.ds(grp * LANES, LANES)]
            for e in range(LANES):
                wb = w16[e]
                ee = grp * LANES + e
                for j in range(D // LANES):
                    rows_v[r4, ee, pl.ds(j * LANES, LANES)] = (
                        rows_v[r4, ee, pl.ds(j * LANES, LANES)] * wb)

    tbl = t_hbm.at[cc]

    for g0 in range(4):
        _stage_start(g0, isems[g0 % 2])
    _stage_wait(sem_i0)
    pltpu.async_copy(tbl.at[src_v.at[0]], rows_v.at[0], sem_g0)
    _stage_wait(sem_i1)
    pltpu.async_copy(tbl.at[src_v.at[1]], rows_v.at[1], sem_g1)

    def _chunk(g, carry):
        par = lax.rem(g, 2)
        r4 = lax.rem(g, 4)
        d8 = lax.rem(g, 8)
        @pl.when(g > 1)
        def _():
            @pl.when(par == 0)
            def _():
                pltpu.make_async_copy(
                    rows_v.at[lax.rem(g + 2, 4)],
                    acc.at[dst_v.at[lax.rem(g + 6, 8)]], sem_s0).wait()
            @pl.when(par == 1)
            def _():
                pltpu.make_async_copy(
                    rows_v.at[lax.rem(g + 2, 4)],
                    acc.at[dst_v.at[lax.rem(g + 6, 8)]], sem_s1).wait()
        @pl.when(par == 0)
        def _():
            pltpu.make_async_copy(tbl.at[src_v.at[r4]], rows_v.at[r4],
                                  sem_g0).wait()
            @pl.when(g + 2 < CHUNKS)
            def _():
                _stage_wait(sem_i0)
                pltpu.async_copy(tbl.at[src_v.at[lax.rem(g + 2, 4)]],
                                 rows_v.at[lax.rem(g + 2, 4)], sem_g0)
        @pl.when(par == 1)
        def _():
            pltpu.make_async_copy(tbl.at[src_v.at[r4]], rows_v.at[r4],
                                  sem_g1).wait()
            @pl.when(g + 2 < CHUNKS)
            def _():
                _stage_wait(sem_i1)
                pltpu.async_copy(tbl.at[src_v.at[lax.rem(g + 2, 4)]],
                                 rows_v.at[lax.rem(g + 2, 4)], sem_g1)
        _compute(r4, r4)
        @pl.when(par == 0)
        def _():
            pltpu.async_copy(rows_v.at[r4], acc.at[dst_v.at[d8]], sem_s0,
                             add=True)
        @pl.when(par == 1)
        def _():
            pltpu.async_copy(rows_v.at[r4], acc.at[dst_v.at[d8]], sem_s1,
                             add=True)
        @pl.when(g + 4 < CHUNKS)
        def _():
            @pl.when(par == 0)
            def _():
                _stage_start(g + 4, sem_i0)
            @pl.when(par == 1)
            def _():
                _stage_start(g + 4, sem_i1)
        return carry

    lax.fori_loop(0, CHUNKS, _chunk, 0)

    pltpu.make_async_copy(rows_v.at[0], acc.at[dst_v.at[0]], sem_s0).wait()
    pltpu.make_async_copy(rows_v.at[1], acc.at[dst_v.at[1]], sem_s1).wait()

    plsc.subcore_barrier()

    off = s * ROWS_PT
    pltpu.sync_copy(acc.at[pl.ds(off, ROWS_PT)],
                    p_hbm.at[cc, pl.ds(off, ROWS_PT)])


def _sc_aggregate(T, se, de, we):
    mesh = plsc.VectorSubcoreMesh(
        core_axis_name="c", subcore_axis_name="s", num_cores=NC, num_subcores=NS)
    f = pl.kernel(
        _sc_body,
        out_type=jax.ShapeDtypeStruct((2, NPAD, D), jnp.float32),
        mesh=mesh,
        scratch_types=[
            pltpu.VMEM((4, CB), jnp.int32),
            pltpu.VMEM((8, CB), jnp.int32),
            pltpu.VMEM((4, CB), jnp.float32),
            pltpu.VMEM((4, CB, D), jnp.float32),
            pltpu.VMEM_SHARED((NPAD, D), jnp.float32),
            pltpu.SemaphoreType.DMA,
            pltpu.SemaphoreType.DMA,
            pltpu.SemaphoreType.DMA,
            pltpu.SemaphoreType.DMA,
            pltpu.SemaphoreType.DMA,
            pltpu.SemaphoreType.DMA,
        ],
    )
    return f(T, se, de, we)


def _combine_body(p0_ref, p1_ref, c_ref, o_ref):
    o_ref[...] = p0_ref[0] + p1_ref[0] + c_ref[...]


def _combine(P, C):
    bm = 1000
    return pl.pallas_call(
        _combine_body,
        grid=(N // bm,),
        in_specs=[
            pl.BlockSpec((1, bm, D), lambda i: (0, i, 0)),
            pl.BlockSpec((1, bm, D), lambda i: (1, i, 0)),
            pl.BlockSpec((bm, D), lambda i: (i, 0)),
        ],
        out_specs=pl.BlockSpec((bm, D), lambda i: (i, 0)),
        out_shape=jax.ShapeDtypeStruct((N, D), jnp.float32),
    )(P, P, C)


def kernel(x, g1_src, g1_dst, g1_w, g2_src, g2_dst, g2_w, W0, W1, merger):
    T, C = _matmuls(x, W0, W1, merger)
    se = jnp.concatenate([g1_src, g2_src])
    de = jnp.concatenate([g1_dst, g2_dst])
    we = jnp.concatenate([g1_w, g2_w])
    P = _sc_aggregate(T, se, de, we)
    return _combine(P, C)
